# baseline (speedup 1.0000x reference)
import jax
import jax.numpy as jnp
from jax.experimental import pallas as pl
from jax.experimental.pallas import tpu as pltpu

_OP_EQ, _OP_LT, _OP_GT = 0, 1, 2
_ALPHA, _BETA = 1.2, 0.7
_B, _D = 8192, 2048
_BM = 512
_NB = _B // _BM
_L = 128


def _loss_body(op_ref, j1_ref, j2_ref, out_ref, acc_ref):
    i = pl.program_id(0)

    j1 = j1_ref[...]
    j2 = j2_ref[...]
    j1b = j1.astype(jnp.bfloat16)
    j2b = j2.astype(jnp.bfloat16)
    ones = jnp.ones((_D, _L), dtype=jnp.bfloat16)
    s1 = jax.lax.dot(j1b * j1b, ones, preferred_element_type=jnp.float32)
    s2 = jax.lax.dot(j2b * j2b, ones, preferred_element_type=jnp.float32)
    s12 = jax.lax.dot(j1b * j2b, ones, preferred_element_type=jnp.float32)

    sd = s1 + s2 - 2.0 * s12
    op = op_ref[...]
    eq = (op == _OP_EQ).astype(jnp.float32)
    lt = (op == _OP_LT).astype(jnp.float32)
    gt = (op == _OP_GT).astype(jnp.float32)
    dn = jnp.sqrt(s1) - jnp.sqrt(s2)
    relu_lt = jnp.maximum(dn, 0.0)
    relu_gt = jnp.maximum(-dn, 0.0)

    @pl.when(i == 0)
    def _init_acc():
        acc_ref[0] = eq * sd
        acc_ref[1] = relu_lt
        acc_ref[2] = relu_gt
        acc_ref[3] = jnp.broadcast_to(eq, (_BM, _L))
        acc_ref[4] = jnp.broadcast_to(lt, (_BM, _L))
        acc_ref[5] = jnp.broadcast_to(gt, (_BM, _L))

    @pl.when(i > 0)
    def _accumulate():
        acc_ref[0] += eq * sd
        acc_ref[1] += relu_lt
        acc_ref[2] += relu_gt
        acc_ref[3] += jnp.broadcast_to(eq, (_BM, _L))
        acc_ref[4] += jnp.broadcast_to(lt, (_BM, _L))
        acc_ref[5] += jnp.broadcast_to(gt, (_BM, _L))

    @pl.when(i == _NB - 1)
    def _finalize():
        inv_l = 1.0 / _L
        eq_sd = jnp.sum(acc_ref[0]) * inv_l
        lt_sum = jnp.sum(acc_ref[1]) * inv_l
        gt_sum = jnp.sum(acc_ref[2]) * inv_l
        eq_cnt = jnp.sum(acc_ref[3]) * inv_l
        lt_cnt = jnp.sum(acc_ref[4]) * inv_l
        gt_cnt = jnp.sum(acc_ref[5]) * inv_l
        has_lt = (lt_cnt > 0.0).astype(jnp.float32)
        has_gt = (gt_cnt > 0.0).astype(jnp.float32)
        eq_loss = eq_sd / jnp.maximum(eq_cnt * _D, 1.0)
        lt_loss = lt_sum * (1.0 / _B)
        gt_loss = gt_sum * (1.0 / _B)
        out_ref[0, 0] = (_ALPHA * eq_loss
                         + _BETA * (has_lt * lt_loss + has_gt * gt_loss))


def kernel(joint1_embedding, joint2_embedding, operation):
    out = pl.pallas_call(
        _loss_body,
        grid=(_NB,),
        in_specs=[
            pl.BlockSpec((_BM, 1), lambda i: (i, 0)),
            pl.BlockSpec((_BM, _D), lambda i: (i, 0)),
            pl.BlockSpec((_BM, _D), lambda i: (i, 0)),
        ],
        out_specs=pl.BlockSpec(memory_space=pltpu.SMEM),
        out_shape=jax.ShapeDtypeStruct((1, 1), jnp.float32),
        scratch_shapes=[
            pltpu.VMEM((6, _BM, _L), jnp.float32),
        ],
    )(operation, joint1_embedding, joint2_embedding)
    return out[0, 0]


# speedup vs baseline: 1.1407x; 1.1407x over previous
import jax
import jax.numpy as jnp
from jax.experimental import pallas as pl
from jax.experimental.pallas import tpu as pltpu

_OP_EQ, _OP_LT, _OP_GT = 0, 1, 2
_ALPHA, _BETA = 1.2, 0.7
_B, _D = 8192, 2048
_BM = 512
_NB = _B // _BM
_L = 128
_TR = 64
_NK = _D // _L
_NT = _BM // _TR


def _loss_body(op_full_ref, op_ref, j1_ref, j2_ref, out_ref, acc_ref,
               stats_ref):
    i = pl.program_id(0)

    @pl.when(i == 0)
    def _init():
        opf = op_full_ref[0, :]
        stats_ref[0] = jnp.sum((opf == _OP_EQ).astype(jnp.float32))
        stats_ref[1] = jnp.sum((opf == _OP_LT).astype(jnp.float32))
        stats_ref[2] = jnp.sum((opf == _OP_GT).astype(jnp.float32))
        acc_ref[...] = jnp.zeros((3, _BM, _L), jnp.float32)

    ones_b = jnp.ones((_L, _L), dtype=jnp.bfloat16)
    for r in range(_NT):
        r0 = r * _TR
        a = j1_ref[r0:r0 + _TR, 0:_L]
        b = j2_ref[r0:r0 + _TR, 0:_L]
        p1 = a * a
        p2 = b * b
        p12 = a * b
        for k in range(1, _NK):
            c0 = k * _L
            a = j1_ref[r0:r0 + _TR, c0:c0 + _L]
            b = j2_ref[r0:r0 + _TR, c0:c0 + _L]
            p1 += a * a
            p2 += b * b
            p12 += a * b
        s1 = jax.lax.dot(p1.astype(jnp.bfloat16), ones_b,
                         preferred_element_type=jnp.float32)
        s2 = jax.lax.dot(p2.astype(jnp.bfloat16), ones_b,
                         preferred_element_type=jnp.float32)
        pd = p1 + p2 - 2.0 * p12
        op_t = op_ref[r0:r0 + _TR, :]
        eq = (op_t == _OP_EQ).astype(jnp.float32)
        dn = jnp.sqrt(s1) - jnp.sqrt(s2)
        acc_ref[0, r0:r0 + _TR, :] += eq * pd
        acc_ref[1, r0:r0 + _TR, :] += jnp.maximum(dn, 0.0)
        acc_ref[2, r0:r0 + _TR, :] += jnp.maximum(-dn, 0.0)

    @pl.when(i == _NB - 1)
    def _finalize():
        inv_l = 1.0 / _L
        eq_sd = jnp.sum(acc_ref[0])
        lt_sum = jnp.sum(acc_ref[1]) * inv_l
        gt_sum = jnp.sum(acc_ref[2]) * inv_l
        eq_cnt = stats_ref[0]
        has_lt = (stats_ref[1] > 0.0).astype(jnp.float32)
        has_gt = (stats_ref[2] > 0.0).astype(jnp.float32)
        eq_loss = eq_sd / jnp.maximum(eq_cnt * _D, 1.0)
        lt_loss = lt_sum * (1.0 / _B)
        gt_loss = gt_sum * (1.0 / _B)
        out_ref[0, 0] = (_ALPHA * eq_loss
                         + _BETA * (has_lt * lt_loss + has_gt * gt_loss))


def kernel(joint1_embedding, joint2_embedding, operation):
    op_row = operation.reshape(1, _B)
    out = pl.pallas_call(
        _loss_body,
        grid=(_NB,),
        in_specs=[
            pl.BlockSpec((1, _B), lambda i: (0, 0)),
            pl.BlockSpec((_BM, 1), lambda i: (i, 0)),
            pl.BlockSpec((_BM, _D), lambda i: (i, 0)),
            pl.BlockSpec((_BM, _D), lambda i: (i, 0)),
        ],
        out_specs=pl.BlockSpec(memory_space=pltpu.SMEM),
        out_shape=jax.ShapeDtypeStruct((1, 1), jnp.float32),
        scratch_shapes=[
            pltpu.VMEM((3, _BM, _L), jnp.float32),
            pltpu.SMEM((3,), jnp.float32),
        ],
    )(op_row, operation, joint1_embedding, joint2_embedding)
    return out[0, 0]


# speedup vs baseline: 1.1507x; 1.0088x over previous
import jax
import jax.numpy as jnp
from jax.experimental import pallas as pl
from jax.experimental.pallas import tpu as pltpu

_OP_EQ, _OP_LT, _OP_GT = 0, 1, 2
_ALPHA, _BETA = 1.2, 0.7
_B, _D = 8192, 2048
_BM = 512
_NB = _B // _BM
_L = 128
_TR = 64
_NK = _D // _L
_NT = _BM // _TR


def _loss_body(op_full_ref, op_ref, j1_ref, j2_ref, out_ref, acc_ref,
               stats_ref):
    i = pl.program_id(0)

    @pl.when(i == 0)
    def _init():
        opf = op_full_ref[0, :]
        stats_ref[0] = jnp.sum((opf == _OP_EQ).astype(jnp.float32))
        stats_ref[1] = jnp.sum((opf == _OP_LT).astype(jnp.float32))
        stats_ref[2] = jnp.sum((opf == _OP_GT).astype(jnp.float32))
        acc_ref[...] = jnp.zeros((3, _TR, _L), jnp.float32)

    ones_b = jnp.ones((_L, _L), dtype=jnp.bfloat16)
    acc0 = acc_ref[0]
    acc1 = acc_ref[1]
    acc2 = acc_ref[2]
    for r in range(_NT):
        r0 = r * _TR
        a = j1_ref[r0:r0 + _TR, 0:_L].astype(jnp.bfloat16)
        b = j2_ref[r0:r0 + _TR, 0:_L].astype(jnp.bfloat16)
        p1 = a * a
        p2 = b * b
        p12 = a * b
        for k in range(1, _NK):
            c0 = k * _L
            a = j1_ref[r0:r0 + _TR, c0:c0 + _L].astype(jnp.bfloat16)
            b = j2_ref[r0:r0 + _TR, c0:c0 + _L].astype(jnp.bfloat16)
            p1 += a * a
            p2 += b * b
            p12 += a * b
        s1 = jax.lax.dot(p1, ones_b, preferred_element_type=jnp.float32)
        s2 = jax.lax.dot(p2, ones_b, preferred_element_type=jnp.float32)
        pd = (p1 + p2 - 2.0 * p12).astype(jnp.float32)
        op_t = op_ref[r0:r0 + _TR, :]
        eq = (op_t == _OP_EQ).astype(jnp.float32)
        dn = jnp.sqrt(s1) - jnp.sqrt(s2)
        acc0 = acc0 + eq * pd
        acc1 = acc1 + jnp.maximum(dn, 0.0)
        acc2 = acc2 + jnp.maximum(-dn, 0.0)
    acc_ref[0] = acc0
    acc_ref[1] = acc1
    acc_ref[2] = acc2

    @pl.when(i == _NB - 1)
    def _finalize():
        inv_l = 1.0 / _L
        eq_sd = jnp.sum(acc_ref[0])
        lt_sum = jnp.sum(acc_ref[1]) * inv_l
        gt_sum = jnp.sum(acc_ref[2]) * inv_l
        eq_cnt = stats_ref[0]
        has_lt = (stats_ref[1] > 0.0).astype(jnp.float32)
        has_gt = (stats_ref[2] > 0.0).astype(jnp.float32)
        eq_loss = eq_sd / jnp.maximum(eq_cnt * _D, 1.0)
        lt_loss = lt_sum * (1.0 / _B)
        gt_loss = gt_sum * (1.0 / _B)
        out_ref[0, 0] = (_ALPHA * eq_loss
                         + _BETA * (has_lt * lt_loss + has_gt * gt_loss))


def kernel(joint1_embedding, joint2_embedding, operation):
    op_row = operation.reshape(1, _B)
    out = pl.pallas_call(
        _loss_body,
        grid=(_NB,),
        in_specs=[
            pl.BlockSpec((1, _B), lambda i: (0, 0)),
            pl.BlockSpec((_BM, 1), lambda i: (i, 0)),
            pl.BlockSpec((_BM, _D), lambda i: (i, 0)),
            pl.BlockSpec((_BM, _D), lambda i: (i, 0)),
        ],
        out_specs=pl.BlockSpec(memory_space=pltpu.SMEM),
        out_shape=jax.ShapeDtypeStruct((1, 1), jnp.float32),
        scratch_shapes=[
            pltpu.VMEM((3, _TR, _L), jnp.float32),
            pltpu.SMEM((3,), jnp.float32),
        ],
    )(op_row, operation, joint1_embedding, joint2_embedding)
    return out[0, 0]
